# baseline (speedup 1.0000x reference)
import functools

import jax
import jax.numpy as jnp
from jax import lax
from jax.experimental import pallas as pl
from jax.experimental.pallas import tpu as pltpu
from jax.experimental.pallas import tpu_sc as plsc

_B, _L = 16384, 200
_N = _B * _L
_V, _D = 100, 10
_VG = 7
_TPAD = _VG * 16 * _D

_INFO = plsc.get_sparse_core_info()
_NC, _NS = _INFO.num_cores, _INFO.num_subcores
_NW = _NC * _NS
_TR = _L // 8
_TC = _B // 128
_JS = 16
_NU = _TR * (_TC // _JS)
_NROUND = 7


def _lut(tab_v, w_v, scores_v):
    b_vec = w_v[pl.ds(_D * 16, 16)]

    def _group(g, c):
        vid = lax.iota(jnp.int32, 16) + g * 16
        acc = b_vec
        for d in range(_D):
            col = plsc.load_gather(tab_v, [vid * _D + d])
            acc = acc + col * w_v[pl.ds(d * 16, 16)]
        scores_v[pl.ds(g * 16, 16)] = acc
        return c

    lax.fori_loop(0, _VG, _group, 0)


def _body(xq_hbm, tab_hbm, w_hbm, out_hbm,
          tab_v, w_v, scores_v, idx0_v, idx1_v, res0_v, res1_v,
          in0_sem, in1_sem, out0_sem, out1_sem):
    t = lax.axis_index("s") * _NC + lax.axis_index("c")

    idx_bufs = (idx0_v, idx1_v)
    res_bufs = (res0_v, res1_v)
    in_sems = (in0_sem, in1_sem)
    out_sems = (out0_sem, out1_sem)

    def in_cp(k, bi):
        u = jnp.minimum(t + _NW * k, _NU - 1)
        return pltpu.make_async_copy(
            xq_hbm.at[u >> 3, pl.ds((u & 7) * _JS, _JS), :, :],
            idx_bufs[bi], in_sems[bi])

    def out_cp(k, bi):
        u = jnp.minimum(t + _NW * k, _NU - 1)
        return pltpu.make_async_copy(
            res_bufs[bi],
            out_hbm.at[pl.ds((u >> 3) * 8, 8), pl.ds((u & 7) * _JS, _JS), :],
            out_sems[bi])

    in_cp(0, 0).start()
    pltpu.sync_copy(tab_hbm, tab_v)
    pltpu.sync_copy(w_hbm, w_v)
    _lut(tab_v, w_v, scores_v)

    def _round(k, bi):
        @pl.when(k + 1 < _NROUND)
        def _():
            in_cp(k + 1, 1 - bi).start()
        in_cp(k, bi).wait()

        @pl.when(k >= 2)
        def _():
            out_cp(k - 2, bi).wait()
        idx_v = idx_bufs[bi]
        res_v = res_bufs[bi]

        @plsc.parallel_loop(0, _JS * 8, step=1, unroll=2)
        def _gather(i):
            j = i >> 3
            s = i & 7
            for l in range(8):
                idx = idx_v[j, s, pl.ds(l * 16, 16)]
                res_v[s, j, pl.ds(l * 16, 16)] = (
                    plsc.load_gather(scores_v, [idx]))

        @pl.when(t + _NW * k < _NU)
        def _():
            out_cp(k, bi).start()

    def _step(k, c):
        @pl.when(k % 2 == 0)
        def _():
            _round(k, 0)

        @pl.when(k % 2 == 1)
        def _():
            _round(k, 1)
        return c

    lax.fori_loop(0, _NROUND, _step, 0)

    for k in range(_NROUND - 2, _NROUND):
        @pl.when(t + _NW * k < _NU)
        def _():
            out_cp(k, k % 2).wait()


@jax.jit
def _run(xq, tab_flat, wb):
    mesh = plsc.VectorSubcoreMesh(core_axis_name="c", subcore_axis_name="s")
    kfn = pl.kernel(
        _body,
        out_type=jax.ShapeDtypeStruct((_L, _TC, 128), jnp.float32),
        mesh=mesh,
        compiler_params=pltpu.CompilerParams(
            needs_layout_passes=False,
            disable_bounds_checks=True,
            disable_semaphore_checks=True,
        ),
        scratch_types=[
            pltpu.VMEM((_TPAD,), jnp.float32),
            pltpu.VMEM(((_D + 1) * 16,), jnp.float32),
            pltpu.VMEM((_VG * 16,), jnp.float32),
            pltpu.VMEM((_JS, 8, 128), jnp.int32),
            pltpu.VMEM((_JS, 8, 128), jnp.int32),
            pltpu.VMEM((8, _JS, 128), jnp.float32),
            pltpu.VMEM((8, _JS, 128), jnp.float32),
            pltpu.SemaphoreType.DMA,
            pltpu.SemaphoreType.DMA,
            pltpu.SemaphoreType.DMA,
            pltpu.SemaphoreType.DMA,
        ],
    )
    return kfn(xq, tab_flat, wb)


def kernel(x, embed_table, lin_w, lin_b):
    xq = (x.astype(jnp.int32).T
          .reshape(_TR, 8, _TC, 128)
          .transpose(0, 2, 1, 3))
    tab_flat = jnp.pad(embed_table.reshape(-1), (0, _TPAD - _V * _D))
    wb = jnp.concatenate([
        jnp.repeat(lin_w.reshape(-1), 16),
        jnp.broadcast_to(lin_b, (16,)),
    ])
    out_t = _run(xq, tab_flat, wb).reshape(_L, _B, 1)
    return out_t.transpose(1, 0, 2)


# speedup vs baseline: 1.0096x; 1.0096x over previous
import functools

import jax
import jax.numpy as jnp
from jax import lax
from jax.experimental import pallas as pl
from jax.experimental.pallas import tpu as pltpu
from jax.experimental.pallas import tpu_sc as plsc

_B, _L = 16384, 200
_N = _B * _L
_V, _D = 100, 10
_VG = 7
_TPAD = _VG * 16 * _D

_INFO = plsc.get_sparse_core_info()
_NC, _NS = _INFO.num_cores, _INFO.num_subcores
_NW = _NC * _NS
_TR = _L // 8
_TC = _B // 128
_JS = 16
_NU = _TR * (_TC // _JS)
_NROUND = 7


def _lut(tab_v, w_v, scores_v):
    b_vec = w_v[pl.ds(_D * 16, 16)]

    def _group(g, c):
        vid = lax.iota(jnp.int32, 16) + g * 16
        acc = b_vec
        for d in range(_D):
            col = plsc.load_gather(tab_v, [vid * _D + d])
            acc = acc + col * w_v[pl.ds(d * 16, 16)]
        scores_v[pl.ds(g * 16, 16)] = acc
        return c

    lax.fori_loop(0, _VG, _group, 0)


def _body(xq_hbm, tab_hbm, w_hbm, out_hbm,
          tab_v, w_v, scores_v, idx0_v, idx1_v, res0_v, res1_v,
          in0_sem, in1_sem, out0_sem, out1_sem):
    t = lax.axis_index("s") * _NC + lax.axis_index("c")

    idx_bufs = (idx0_v, idx1_v)
    res_bufs = (res0_v, res1_v)
    in_sems = (in0_sem, in1_sem)
    out_sems = (out0_sem, out1_sem)

    def in_cp(k, bi):
        u = jnp.minimum(t + _NW * k, _NU - 1)
        return pltpu.make_async_copy(
            xq_hbm.at[u >> 3, pl.ds((u & 7) * _JS, _JS), :, :],
            idx_bufs[bi], in_sems[bi])

    def out_cp(k, bi):
        u = jnp.minimum(t + _NW * k, _NU - 1)
        return pltpu.make_async_copy(
            res_bufs[bi],
            out_hbm.at[pl.ds((u >> 3) * 8, 8), pl.ds((u & 7) * _JS, _JS), :],
            out_sems[bi])

    in_cp(0, 0).start()
    pltpu.sync_copy(tab_hbm, tab_v)
    pltpu.sync_copy(w_hbm, w_v)
    _lut(tab_v, w_v, scores_v)

    def _round(k, bi):
        @pl.when(k + 1 < _NROUND)
        def _():
            in_cp(k + 1, 1 - bi).start()
        in_cp(k, bi).wait()

        @pl.when(k >= 2)
        def _():
            out_cp(k - 2, bi).wait()
        idx_v = idx_bufs[bi]
        res_v = res_bufs[bi]

        @plsc.parallel_loop(0, _JS * 8, step=1)
        def _gather(i):
            j = i >> 3
            s = i & 7
            for l in range(8):
                idx = idx_v[j, s, pl.ds(l * 16, 16)]
                res_v[s, j, pl.ds(l * 16, 16)] = (
                    plsc.load_gather(scores_v, [idx]))

        @pl.when(t + _NW * k < _NU)
        def _():
            out_cp(k, bi).start()

    def _step(k, c):
        @pl.when(k % 2 == 0)
        def _():
            _round(k, 0)

        @pl.when(k % 2 == 1)
        def _():
            _round(k, 1)
        return c

    lax.fori_loop(0, _NROUND, _step, 0)

    for k in range(_NROUND - 2, _NROUND):
        @pl.when(t + _NW * k < _NU)
        def _():
            out_cp(k, k % 2).wait()


@jax.jit
def _run(xq, tab_flat, wb):
    mesh = plsc.VectorSubcoreMesh(core_axis_name="c", subcore_axis_name="s")
    kfn = pl.kernel(
        _body,
        out_type=jax.ShapeDtypeStruct((_L, _TC, 128), jnp.float32),
        mesh=mesh,
        compiler_params=pltpu.CompilerParams(
            needs_layout_passes=False,
            disable_bounds_checks=True,
            disable_semaphore_checks=True,
        ),
        scratch_types=[
            pltpu.VMEM((_TPAD,), jnp.float32),
            pltpu.VMEM(((_D + 1) * 16,), jnp.float32),
            pltpu.VMEM((_VG * 16,), jnp.float32),
            pltpu.VMEM((_JS, 8, 128), jnp.int32),
            pltpu.VMEM((_JS, 8, 128), jnp.int32),
            pltpu.VMEM((8, _JS, 128), jnp.float32),
            pltpu.VMEM((8, _JS, 128), jnp.float32),
            pltpu.SemaphoreType.DMA,
            pltpu.SemaphoreType.DMA,
            pltpu.SemaphoreType.DMA,
            pltpu.SemaphoreType.DMA,
        ],
    )
    return kfn(xq, tab_flat, wb)


def kernel(x, embed_table, lin_w, lin_b):
    xq = (x.astype(jnp.int32).T
          .reshape(_TR, 8, _TC, 128)
          .transpose(0, 2, 1, 3))
    tab_flat = jnp.pad(embed_table.reshape(-1), (0, _TPAD - _V * _D))
    wb = jnp.concatenate([
        jnp.repeat(lin_w.reshape(-1), 16),
        jnp.broadcast_to(lin_b, (16,)),
    ])
    out_t = _run(xq, tab_flat, wb).reshape(_L, _B, 1)
    return out_t.transpose(1, 0, 2)
